# baseline (speedup 1.0000x reference)
import jax
import jax.numpy as jnp
from jax import lax
from jax.experimental import pallas as pl
from jax.experimental.pallas import tpu as pltpu
from jax.experimental.pallas import tpu_sc as plsc

N = 10000
E = 320000
D = 128
G = 500
P = 4096
RH = 256

NC, NS = 2, 16
NW = NC * NS
NP = 10240
EP = 327680
GP = 512
ECH = 80
NCH = 5
RPT = NP // NS

_mesh = plsc.VectorSubcoreMesh(core_axis_name="c", subcore_axis_name="s")
_f32 = jnp.float32


def _zero_rows(ref, n16):
    z = jnp.zeros((16,), _f32)
    def row(i, _):
        for j in range(8):
            ref[i, pl.ds(j * 16, 16)] = z
        return 0
    lax.fori_loop(0, 16 * n16, row, 0)


def _fill_1d(ref, n, val):
    v = jnp.full((16,), val, _f32)
    def it(i, _):
        ref[pl.ds(i * 16, 16)] = v
        return 0
    lax.fori_loop(0, n // 16, it, 0)


def _hist_body(dst_hbm, bat_hbm, deg_out, cnt_out,
               didx_v, bidx_v, ones_v, zb_v, deg_sh, cnt_sh):
    cid = lax.axis_index("c")
    sid = lax.axis_index("s")
    wid = cid * NS + sid
    _fill_1d(ones_v, 128, 1.0)
    _fill_1d(zb_v, RPT, 0.0)
    pltpu.sync_copy(zb_v, deg_sh.at[pl.ds(sid * RPT, RPT)])
    pltpu.sync_copy(zb_v.at[pl.ds(0, GP // NS)],
                    cnt_sh.at[pl.ds(sid * (GP // NS), GP // NS)])
    plsc.subcore_barrier()
    pltpu.sync_copy(dst_hbm.at[pl.ds(wid * ECH, ECH)], didx_v)
    def echunk(c, _):
        pltpu.sync_copy(ones_v, deg_sh.at[didx_v.at[c]], add=True)
        return 0
    lax.fori_loop(0, ECH, echunk, 0)
    pltpu.sync_copy(bat_hbm.at[wid], bidx_v)
    def bchunk(c, _):
        pltpu.sync_copy(ones_v.at[pl.ds(0, 64)], cnt_sh.at[bidx_v.at[c]],
                        add=True)
        return 0
    lax.fori_loop(0, NCH, bchunk, 0)
    plsc.subcore_barrier()
    pltpu.sync_copy(deg_sh.at[pl.ds(sid * RPT, RPT)], zb_v)
    pltpu.sync_copy(zb_v, deg_out.at[pl.ds(cid * NP + sid * RPT, RPT)])
    pltpu.sync_copy(cnt_sh.at[pl.ds(sid * (GP // NS), GP // NS)],
                    ones_v.at[pl.ds(0, GP // NS)])
    pltpu.sync_copy(ones_v.at[pl.ds(0, GP // NS)],
                    cnt_out.at[pl.ds(cid * GP + sid * (GP // NS), GP // NS)])


_hist = pl.kernel(
    _hist_body,
    out_type=(jax.ShapeDtypeStruct((NC * NP,), _f32),
              jax.ShapeDtypeStruct((NC * GP,), _f32)),
    mesh=_mesh,
    scratch_types=[
        pltpu.VMEM((ECH, 128), jnp.int32),
        pltpu.VMEM((NCH, 64), jnp.int32),
        pltpu.VMEM((128,), _f32),
        pltpu.VMEM((RPT,), _f32),
        pltpu.VMEM_SHARED((NP,), _f32),
        pltpu.VMEM_SHARED((GP,), _f32),
    ],
)


CW = 64
NBUF = 4
NGIF = 2
ECW = EP // NW // CW
PH = 4
PCH = ECW // PH


def _prop_body(q_hbm, src_hbm, dst_hbm, out_hbm,
               sidx_v, didx_v, rows0, rows1, rows2, rows3, acc_sh,
               gs0, gs1, gs2, gs3, ss0, ss1, ss2, ss3):
    cid = lax.axis_index("c")
    sid = lax.axis_index("s")
    wid = cid * NS + sid
    rows = (rows0, rows1, rows2, rows3)
    gsem = (gs0, gs1, gs2, gs3)
    ssem = (ss0, ss1, ss2, ss3)
    _zero_rows(rows0, 1)
    def zrow(i, _):
        pltpu.sync_copy(rows0.at[pl.ds(0, 16)],
                        acc_sh.at[pl.ds(sid * RPT + i * 16, 16)])
        return 0
    lax.fori_loop(0, RPT // 16, zrow, 0)
    plsc.subcore_barrier()

    def wait_bytes(sem, buf):
        pltpu.make_async_copy(q_hbm.at[pl.ds(0, CW)], buf, sem).wait()

    def phase(p, _):
        pltpu.sync_copy(src_hbm.at[wid, pl.ds(p * PCH, PCH)], sidx_v)
        pltpu.sync_copy(dst_hbm.at[wid, pl.ds(p * PCH, PCH)], didx_v)
        for j in range(NGIF):
            pltpu.async_copy(q_hbm.at[sidx_v.at[j]], rows[j], gsem[j])
        def step(g, _):
            for b in range(NBUF):
                c = NBUF * g + b
                bj = (b + NGIF) % NBUF
                wait_bytes(gsem[b], rows[b])
                pltpu.async_copy(rows[b], acc_sh.at[didx_v.at[c]],
                                 ssem[b], add=True)
                @pl.when(c + NGIF < PCH)
                def _():
                    @pl.when(c >= NBUF - NGIF)
                    def _():
                        wait_bytes(ssem[bj], rows[bj])
                    pltpu.async_copy(q_hbm.at[sidx_v.at[c + NGIF]],
                                     rows[bj], gsem[bj])
            return 0
        lax.fori_loop(0, PCH // NBUF, step, 0)
        for b in range(NBUF):
            wait_bytes(ssem[b], rows[b])
        return 0
    lax.fori_loop(0, PH, phase, 0)
    plsc.subcore_barrier()
    def rb(i, _):
        pltpu.sync_copy(acc_sh.at[pl.ds(sid * RPT + i * CW, CW)], rows0)
        pltpu.sync_copy(rows0,
                        out_hbm.at[pl.ds(cid * NP + sid * RPT + i * CW, CW)])
        return 0
    lax.fori_loop(0, RPT // CW, rb, 0)


_prop = pl.kernel(
    _prop_body,
    out_type=jax.ShapeDtypeStruct((NC * NP, D), _f32),
    mesh=_mesh,
    scratch_types=[
        pltpu.VMEM((PCH, CW), jnp.int32),
        pltpu.VMEM((PCH, CW), jnp.int32),
        pltpu.VMEM((CW, D), _f32),
        pltpu.VMEM((CW, D), _f32),
        pltpu.VMEM((CW, D), _f32),
        pltpu.VMEM((CW, D), _f32),
        pltpu.VMEM_SHARED((NP, D), _f32),
        pltpu.SemaphoreType.DMA,
        pltpu.SemaphoreType.DMA,
        pltpu.SemaphoreType.DMA,
        pltpu.SemaphoreType.DMA,
        pltpu.SemaphoreType.DMA,
        pltpu.SemaphoreType.DMA,
        pltpu.SemaphoreType.DMA,
        pltpu.SemaphoreType.DMA,
    ],
)


def _pool_body(h_hbm, bat_hbm, out_hbm, bidx_v, rows_v, zr_v, g_sh):
    cid = lax.axis_index("c")
    sid = lax.axis_index("s")
    wid = cid * NS + sid
    _zero_rows(zr_v, 1)
    def zrow(i, _):
        pltpu.sync_copy(zr_v, g_sh.at[pl.ds(sid * (GP // NS) + i * 16, 16)])
        return 0
    lax.fori_loop(0, GP // NS // 16, zrow, 0)
    plsc.subcore_barrier()
    pltpu.sync_copy(bat_hbm.at[wid], bidx_v)
    def chunk(c, _):
        pltpu.sync_copy(h_hbm.at[pl.ds(wid * (NCH * 64) + c * 64, 64)], rows_v)
        pltpu.sync_copy(rows_v, g_sh.at[bidx_v.at[c]], add=True)
        return 0
    lax.fori_loop(0, NCH, chunk, 0)
    plsc.subcore_barrier()
    pltpu.sync_copy(g_sh.at[pl.ds(sid * (GP // NS), GP // NS)],
                    rows_v.at[pl.ds(0, GP // NS)])
    pltpu.sync_copy(rows_v.at[pl.ds(0, GP // NS)],
                    out_hbm.at[pl.ds(cid * GP + sid * (GP // NS), GP // NS)])


_pool = pl.kernel(
    _pool_body,
    out_type=jax.ShapeDtypeStruct((NC * GP, D), _f32),
    mesh=_mesh,
    scratch_types=[
        pltpu.VMEM((NCH, 64), jnp.int32),
        pltpu.VMEM((64, D), _f32),
        pltpu.VMEM((16, D), _f32),
        pltpu.VMEM_SHARED((GP, D), _f32),
    ],
)


def _pairs_body(tab_hbm, pidx_hbm, out_hbm, pidx_v, rows_v, sem):
    cid = lax.axis_index("c")
    sid = lax.axis_index("s")
    wid = cid * NS + sid
    pltpu.sync_copy(pidx_hbm.at[wid], pidx_v)
    def chunk(c, _):
        pltpu.async_copy(tab_hbm.at[pidx_v.at[c]], rows_v, sem).wait()
        pltpu.sync_copy(rows_v, out_hbm.at[pl.ds(wid * 256 + c * 128, 128)])
        return 0
    lax.fori_loop(0, 2, chunk, 0)


_pairs = pl.kernel(
    _pairs_body,
    out_type=jax.ShapeDtypeStruct((2 * P, D), _f32),
    mesh=_mesh,
    scratch_types=[
        pltpu.VMEM((2, 128), jnp.int32),
        pltpu.VMEM((128, D), _f32),
        pltpu.SemaphoreType.DMA,
    ],
)


_RB = 1024


def _tc1_body(x_ref, w_ref, deg_ref, q_ref, dis_ref):
    deg = deg_ref[0] + deg_ref[1] + 1.0
    dis = lax.rsqrt(deg)
    h = jnp.dot(x_ref[...], w_ref[...], preferred_element_type=_f32, precision=lax.Precision.HIGHEST)
    q_ref[...] = h * dis
    dis_ref[...] = dis


def _tc1(x_pad, W1, deg3):
    return pl.pallas_call(
        _tc1_body,
        grid=(NP // _RB,),
        in_specs=[
            pl.BlockSpec((_RB, D), lambda i: (i, 0)),
            pl.BlockSpec((D, D), lambda i: (0, 0)),
            pl.BlockSpec((NC, _RB, 1), lambda i: (0, i, 0)),
        ],
        out_specs=[
            pl.BlockSpec((_RB, D), lambda i: (i, 0)),
            pl.BlockSpec((_RB, 1), lambda i: (i, 0)),
        ],
        out_shape=[jax.ShapeDtypeStruct((NP, D), _f32),
                   jax.ShapeDtypeStruct((NP, 1), _f32)],
    )(x_pad, W1, deg3)


def _tc2_body(acc_ref, q_ref, dis_ref, b_ref, w_ref, oq_ref):
    s = acc_ref[0] + acc_ref[1] + q_ref[...]
    o1 = jnp.maximum(s * dis_ref[...] + b_ref[...], 0.0)
    oq_ref[...] = jnp.dot(o1, w_ref[...],
                          preferred_element_type=_f32, precision=lax.Precision.HIGHEST) * dis_ref[...]


def _tc2(acc3, q1, dis, b1r, W2):
    return pl.pallas_call(
        _tc2_body,
        grid=(NP // _RB,),
        in_specs=[
            pl.BlockSpec((NC, _RB, D), lambda i: (0, i, 0)),
            pl.BlockSpec((_RB, D), lambda i: (i, 0)),
            pl.BlockSpec((_RB, 1), lambda i: (i, 0)),
            pl.BlockSpec((1, D), lambda i: (0, 0)),
            pl.BlockSpec((D, D), lambda i: (0, 0)),
        ],
        out_specs=pl.BlockSpec((_RB, D), lambda i: (i, 0)),
        out_shape=jax.ShapeDtypeStruct((NP, D), _f32),
    )(acc3, q1, dis, b1r, W2)


def _tc3_body(acc_ref, q_ref, dis_ref, b_ref, h_ref):
    s = acc_ref[0] + acc_ref[1] + q_ref[...]
    h_ref[...] = jnp.maximum(s * dis_ref[...] + b_ref[...], 0.0)


def _tc3(acc3, q2, dis, b2r):
    return pl.pallas_call(
        _tc3_body,
        grid=(NP // _RB,),
        in_specs=[
            pl.BlockSpec((NC, _RB, D), lambda i: (0, i, 0)),
            pl.BlockSpec((_RB, D), lambda i: (i, 0)),
            pl.BlockSpec((_RB, 1), lambda i: (i, 0)),
            pl.BlockSpec((1, D), lambda i: (0, 0)),
        ],
        out_specs=pl.BlockSpec((_RB, D), lambda i: (i, 0)),
        out_shape=jax.ShapeDtypeStruct((NP, D), _f32),
    )(acc3, q2, dis, b2r)


def _tc4_body(gs_ref, cnt_ref, ge_ref):
    cnt = cnt_ref[0] + cnt_ref[1]
    ge_ref[...] = (gs_ref[0] + gs_ref[1]) / jnp.maximum(cnt, 1.0)


def _tc4(gs3, cnt3):
    return pl.pallas_call(
        _tc4_body,
        out_shape=jax.ShapeDtypeStruct((GP, D), _f32),
    )(gs3, cnt3)


def _tc5_body(zf_ref, zt_ref, w1_ref, b1_ref, w2_ref, b2_ref, o_ref):
    z = (jnp.dot(zf_ref[...], w1_ref[0:D, :],
                 preferred_element_type=_f32, precision=lax.Precision.HIGHEST)
         + jnp.dot(zt_ref[...], w1_ref[D:2 * D, :],
                   preferred_element_type=_f32, precision=lax.Precision.HIGHEST))
    o = jnp.maximum(z + b1_ref[...], 0.0)
    o_ref[...] = jnp.dot(o, w2_ref[...], preferred_element_type=_f32, precision=lax.Precision.HIGHEST) + b2_ref[...]


def _tc5(zg, Wr1, br1r, Wr2, br2r):
    blk = 512
    return pl.pallas_call(
        _tc5_body,
        grid=(P // blk,),
        in_specs=[
            pl.BlockSpec((blk, D), lambda i: (i, 0)),
            pl.BlockSpec((blk, D), lambda i: (i + P // blk, 0)),
            pl.BlockSpec((2 * D, RH), lambda i: (0, 0)),
            pl.BlockSpec((1, RH), lambda i: (0, 0)),
            pl.BlockSpec((RH, 1), lambda i: (0, 0)),
            pl.BlockSpec((1, 1), lambda i: (0, 0)),
        ],
        out_specs=pl.BlockSpec((blk, 1), lambda i: (i, 0)),
        out_shape=jax.ShapeDtypeStruct((P, 1), _f32),
    )(zg, zg, Wr1, br1r, Wr2, br2r)


def kernel(x, edge_index, batch, drug_drug_batch,
           W1, b1, W2, b2, Wr1, br1, Wr2, br2):
    pad_e = N + jnp.arange(EP - E, dtype=jnp.int32) % (NP - N)
    src_f = jnp.concatenate([edge_index[0], pad_e])
    dst_f = jnp.concatenate([edge_index[1], pad_e])
    src = src_f.reshape(NW, ECW, CW)
    dst = dst_f.reshape(NW, ECW, CW)
    dsth = dst_f.reshape(NW * ECH, 128)
    bat = jnp.concatenate([batch, jnp.full((NP - N,), G, jnp.int32)])
    bat = bat.reshape(NW, NCH, 64)
    x_pad = jnp.pad(x, ((0, NP - N), (0, 0)))

    degf, cntf = _hist(dsth, bat)
    q1, dis = _tc1(x_pad, W1, degf.reshape(NC, NP, 1))
    acc1 = _prop(q1, src, dst)
    q2 = _tc2(acc1.reshape(NC, NP, D), q1, dis, b1.reshape(1, D), W2)
    acc2 = _prop(q2, src, dst)
    h = _tc3(acc2.reshape(NC, NP, D), q2, dis, b2.reshape(1, D))
    gs = _pool(h, bat)
    ge = _tc4(gs.reshape(NC, GP, D), cntf.reshape(NC, GP, 1))
    pidx = jnp.concatenate([drug_drug_batch[0],
                            drug_drug_batch[1]]).reshape(NW, 2, 128)
    zg = _pairs(ge, pidx)
    return _tc5(zg, Wr1, br1.reshape(1, RH), Wr2, br2.reshape(1, 1))


# speedup vs baseline: 1.0115x; 1.0115x over previous
import jax
import jax.numpy as jnp
from jax import lax
from jax.experimental import pallas as pl
from jax.experimental.pallas import tpu as pltpu
from jax.experimental.pallas import tpu_sc as plsc

N = 10000
E = 320000
D = 128
G = 500
P = 4096
RH = 256

NC, NS = 2, 16
NW = NC * NS
NP = 10240
EP = 327680
GP = 512
ECH = 80
NCH = 5
RPT = NP // NS

_mesh = plsc.VectorSubcoreMesh(core_axis_name="c", subcore_axis_name="s")
_f32 = jnp.float32


def _zero_rows(ref, n16):
    z = jnp.zeros((16,), _f32)
    def row(i, _):
        for j in range(8):
            ref[i, pl.ds(j * 16, 16)] = z
        return 0
    lax.fori_loop(0, 16 * n16, row, 0)


def _fill_1d(ref, n, val):
    v = jnp.full((16,), val, _f32)
    def it(i, _):
        ref[pl.ds(i * 16, 16)] = v
        return 0
    lax.fori_loop(0, n // 16, it, 0)


def _hist_body(dst_hbm, bat_hbm, deg_out, cnt_out,
               didx_v, bidx_v, ones_v, zb_v, deg_sh, cnt_sh):
    cid = lax.axis_index("c")
    sid = lax.axis_index("s")
    wid = cid * NS + sid
    _fill_1d(ones_v, 128, 1.0)
    _fill_1d(zb_v, RPT, 0.0)
    pltpu.sync_copy(zb_v, deg_sh.at[pl.ds(sid * RPT, RPT)])
    pltpu.sync_copy(zb_v.at[pl.ds(0, GP // NS)],
                    cnt_sh.at[pl.ds(sid * (GP // NS), GP // NS)])
    plsc.subcore_barrier()
    pltpu.sync_copy(dst_hbm.at[pl.ds(wid * ECH, ECH)], didx_v)
    def echunk(c, _):
        pltpu.sync_copy(ones_v, deg_sh.at[didx_v.at[c]], add=True)
        return 0
    lax.fori_loop(0, ECH, echunk, 0)
    pltpu.sync_copy(bat_hbm.at[wid], bidx_v)
    def bchunk(c, _):
        pltpu.sync_copy(ones_v.at[pl.ds(0, 64)], cnt_sh.at[bidx_v.at[c]],
                        add=True)
        return 0
    lax.fori_loop(0, NCH, bchunk, 0)
    plsc.subcore_barrier()
    pltpu.sync_copy(deg_sh.at[pl.ds(sid * RPT, RPT)], zb_v)
    pltpu.sync_copy(zb_v, deg_out.at[pl.ds(cid * NP + sid * RPT, RPT)])
    pltpu.sync_copy(cnt_sh.at[pl.ds(sid * (GP // NS), GP // NS)],
                    ones_v.at[pl.ds(0, GP // NS)])
    pltpu.sync_copy(ones_v.at[pl.ds(0, GP // NS)],
                    cnt_out.at[pl.ds(cid * GP + sid * (GP // NS), GP // NS)])


_hist = pl.kernel(
    _hist_body,
    out_type=(jax.ShapeDtypeStruct((NC * NP,), _f32),
              jax.ShapeDtypeStruct((NC * GP,), _f32)),
    mesh=_mesh,
    scratch_types=[
        pltpu.VMEM((ECH, 128), jnp.int32),
        pltpu.VMEM((NCH, 64), jnp.int32),
        pltpu.VMEM((128,), _f32),
        pltpu.VMEM((RPT,), _f32),
        pltpu.VMEM_SHARED((NP,), _f32),
        pltpu.VMEM_SHARED((GP,), _f32),
    ],
)


CW = 64
NBUF = 4
NGIF = 2
ECW = EP // NW // CW
PH = 4
PCH = ECW // PH


def _prop_body(q_hbm, src_hbm, dst_hbm, out_hbm,
               sidx_v, didx_v, rows0, rows1, rows2, rows3, acc_sh,
               gs0, gs1, gs2, gs3, ss0, ss1, ss2, ss3):
    cid = lax.axis_index("c")
    sid = lax.axis_index("s")
    wid = cid * NS + sid
    rows = (rows0, rows1, rows2, rows3)
    gsem = (gs0, gs1, gs2, gs3)
    ssem = (ss0, ss1, ss2, ss3)
    _zero_rows(rows0, 1)
    def zrow(i, _):
        pltpu.sync_copy(rows0.at[pl.ds(0, 16)],
                        acc_sh.at[pl.ds(sid * RPT + i * 16, 16)])
        return 0
    lax.fori_loop(0, RPT // 16, zrow, 0)
    plsc.subcore_barrier()

    def wait_bytes(sem, buf):
        pltpu.make_async_copy(q_hbm.at[pl.ds(0, CW)], buf, sem).wait()

    def phase(p, _):
        pltpu.sync_copy(src_hbm.at[wid, pl.ds(p * PCH, PCH)], sidx_v)
        pltpu.sync_copy(dst_hbm.at[wid, pl.ds(p * PCH, PCH)], didx_v)
        for j in range(NGIF):
            pltpu.async_copy(q_hbm.at[sidx_v.at[j]], rows[j], gsem[j])
        def step(g, _):
            for b in range(NBUF):
                c = NBUF * g + b
                bj = (b + NGIF) % NBUF
                wait_bytes(gsem[b], rows[b])
                pltpu.async_copy(rows[b], acc_sh.at[didx_v.at[c]],
                                 ssem[b], add=True)
                @pl.when(c + NGIF < PCH)
                def _():
                    @pl.when(c >= NBUF - NGIF)
                    def _():
                        wait_bytes(ssem[bj], rows[bj])
                    pltpu.async_copy(q_hbm.at[sidx_v.at[c + NGIF]],
                                     rows[bj], gsem[bj])
            return 0
        lax.fori_loop(0, PCH // NBUF, step, 0)
        for b in range(NBUF):
            wait_bytes(ssem[b], rows[b])
        return 0
    lax.fori_loop(0, PH, phase, 0)
    plsc.subcore_barrier()
    def rb(i, _):
        pltpu.sync_copy(acc_sh.at[pl.ds(sid * RPT + i * CW, CW)], rows0)
        pltpu.sync_copy(rows0,
                        out_hbm.at[pl.ds(cid * NP + sid * RPT + i * CW, CW)])
        return 0
    lax.fori_loop(0, RPT // CW, rb, 0)


_prop = pl.kernel(
    _prop_body,
    out_type=jax.ShapeDtypeStruct((NC * NP, D), _f32),
    mesh=_mesh,
    scratch_types=[
        pltpu.VMEM((PCH, CW), jnp.int32),
        pltpu.VMEM((PCH, CW), jnp.int32),
        pltpu.VMEM((CW, D), _f32),
        pltpu.VMEM((CW, D), _f32),
        pltpu.VMEM((CW, D), _f32),
        pltpu.VMEM((CW, D), _f32),
        pltpu.VMEM_SHARED((NP, D), _f32),
        pltpu.SemaphoreType.DMA,
        pltpu.SemaphoreType.DMA,
        pltpu.SemaphoreType.DMA,
        pltpu.SemaphoreType.DMA,
        pltpu.SemaphoreType.DMA,
        pltpu.SemaphoreType.DMA,
        pltpu.SemaphoreType.DMA,
        pltpu.SemaphoreType.DMA,
    ],
)


def _pool_body(h_hbm, bat_hbm, out_hbm, bidx_v, rows_v, zr_v, g_sh):
    cid = lax.axis_index("c")
    sid = lax.axis_index("s")
    wid = cid * NS + sid
    _zero_rows(zr_v, 1)
    def zrow(i, _):
        pltpu.sync_copy(zr_v, g_sh.at[pl.ds(sid * (GP // NS) + i * 16, 16)])
        return 0
    lax.fori_loop(0, GP // NS // 16, zrow, 0)
    plsc.subcore_barrier()
    pltpu.sync_copy(bat_hbm.at[wid], bidx_v)
    def chunk(c, _):
        pltpu.sync_copy(h_hbm.at[pl.ds(wid * (NCH * 64) + c * 64, 64)], rows_v)
        pltpu.sync_copy(rows_v, g_sh.at[bidx_v.at[c]], add=True)
        return 0
    lax.fori_loop(0, NCH, chunk, 0)
    plsc.subcore_barrier()
    pltpu.sync_copy(g_sh.at[pl.ds(sid * (GP // NS), GP // NS)],
                    rows_v.at[pl.ds(0, GP // NS)])
    pltpu.sync_copy(rows_v.at[pl.ds(0, GP // NS)],
                    out_hbm.at[pl.ds(cid * GP + sid * (GP // NS), GP // NS)])


_pool = pl.kernel(
    _pool_body,
    out_type=jax.ShapeDtypeStruct((NC * GP, D), _f32),
    mesh=_mesh,
    scratch_types=[
        pltpu.VMEM((NCH, 64), jnp.int32),
        pltpu.VMEM((64, D), _f32),
        pltpu.VMEM((16, D), _f32),
        pltpu.VMEM_SHARED((GP, D), _f32),
    ],
)


def _pairs_body(tab_hbm, pidx_hbm, out_hbm, pidx_v, rows_v, sem):
    cid = lax.axis_index("c")
    sid = lax.axis_index("s")
    wid = cid * NS + sid
    pltpu.sync_copy(pidx_hbm.at[wid], pidx_v)
    def chunk(c, _):
        pltpu.async_copy(tab_hbm.at[pidx_v.at[c]], rows_v, sem).wait()
        pltpu.sync_copy(rows_v, out_hbm.at[pl.ds(wid * 256 + c * 128, 128)])
        return 0
    lax.fori_loop(0, 2, chunk, 0)


_pairs = pl.kernel(
    _pairs_body,
    out_type=jax.ShapeDtypeStruct((2 * P, RH), _f32),
    mesh=_mesh,
    scratch_types=[
        pltpu.VMEM((2, 128), jnp.int32),
        pltpu.VMEM((128, RH), _f32),
        pltpu.SemaphoreType.DMA,
    ],
)


_RB = 1024


def _tc1_body(x_ref, w_ref, deg_ref, q_ref, dis_ref):
    deg = deg_ref[0] + deg_ref[1] + 1.0
    dis = lax.rsqrt(deg)
    h = jnp.dot(x_ref[...], w_ref[...], preferred_element_type=_f32, precision=lax.Precision.HIGHEST)
    q_ref[...] = h * dis
    dis_ref[...] = dis


def _tc1(x_pad, W1, deg3):
    return pl.pallas_call(
        _tc1_body,
        grid=(NP // _RB,),
        in_specs=[
            pl.BlockSpec((_RB, D), lambda i: (i, 0)),
            pl.BlockSpec((D, D), lambda i: (0, 0)),
            pl.BlockSpec((NC, _RB, 1), lambda i: (0, i, 0)),
        ],
        out_specs=[
            pl.BlockSpec((_RB, D), lambda i: (i, 0)),
            pl.BlockSpec((_RB, 1), lambda i: (i, 0)),
        ],
        out_shape=[jax.ShapeDtypeStruct((NP, D), _f32),
                   jax.ShapeDtypeStruct((NP, 1), _f32)],
    )(x_pad, W1, deg3)


def _tc2_body(acc_ref, q_ref, dis_ref, b_ref, w_ref, oq_ref):
    s = acc_ref[0] + acc_ref[1] + q_ref[...]
    o1 = jnp.maximum(s * dis_ref[...] + b_ref[...], 0.0)
    oq_ref[...] = jnp.dot(o1, w_ref[...],
                          preferred_element_type=_f32, precision=lax.Precision.HIGHEST) * dis_ref[...]


def _tc2(acc3, q1, dis, b1r, W2):
    return pl.pallas_call(
        _tc2_body,
        grid=(NP // _RB,),
        in_specs=[
            pl.BlockSpec((NC, _RB, D), lambda i: (0, i, 0)),
            pl.BlockSpec((_RB, D), lambda i: (i, 0)),
            pl.BlockSpec((_RB, 1), lambda i: (i, 0)),
            pl.BlockSpec((1, D), lambda i: (0, 0)),
            pl.BlockSpec((D, D), lambda i: (0, 0)),
        ],
        out_specs=pl.BlockSpec((_RB, D), lambda i: (i, 0)),
        out_shape=jax.ShapeDtypeStruct((NP, D), _f32),
    )(acc3, q1, dis, b1r, W2)


def _tc3_body(acc_ref, q_ref, dis_ref, b_ref, h_ref):
    s = acc_ref[0] + acc_ref[1] + q_ref[...]
    h_ref[...] = jnp.maximum(s * dis_ref[...] + b_ref[...], 0.0)


def _tc3(acc3, q2, dis, b2r):
    return pl.pallas_call(
        _tc3_body,
        grid=(NP // _RB,),
        in_specs=[
            pl.BlockSpec((NC, _RB, D), lambda i: (0, i, 0)),
            pl.BlockSpec((_RB, D), lambda i: (i, 0)),
            pl.BlockSpec((_RB, 1), lambda i: (i, 0)),
            pl.BlockSpec((1, D), lambda i: (0, 0)),
        ],
        out_specs=pl.BlockSpec((_RB, D), lambda i: (i, 0)),
        out_shape=jax.ShapeDtypeStruct((NP, D), _f32),
    )(acc3, q2, dis, b2r)


def _tc4_body(gs_ref, cnt_ref, w_ref, u_ref, v_ref):
    cnt = cnt_ref[0] + cnt_ref[1]
    ge = (gs_ref[0] + gs_ref[1]) / jnp.maximum(cnt, 1.0)
    u_ref[...] = jnp.dot(ge, w_ref[0:D, :], preferred_element_type=_f32, precision=lax.Precision.HIGHEST)
    v_ref[...] = jnp.dot(ge, w_ref[D:2 * D, :], preferred_element_type=_f32, precision=lax.Precision.HIGHEST)


def _tc4(gs3, cnt3, Wr1):
    return pl.pallas_call(
        _tc4_body,
        out_shape=[jax.ShapeDtypeStruct((GP, RH), _f32),
                   jax.ShapeDtypeStruct((GP, RH), _f32)],
    )(gs3, cnt3, Wr1)


def _tc5_body(zf_ref, zt_ref, b1_ref, w_ref, b2_ref, o_ref):
    o = jnp.maximum(zf_ref[...] + zt_ref[...] + b1_ref[...], 0.0)
    o_ref[...] = jnp.dot(o, w_ref[...], preferred_element_type=_f32, precision=lax.Precision.HIGHEST) + b2_ref[...]


def _tc5(zg, br1r, Wr2, br2r):
    blk = 512
    return pl.pallas_call(
        _tc5_body,
        grid=(P // blk,),
        in_specs=[
            pl.BlockSpec((blk, RH), lambda i: (i, 0)),
            pl.BlockSpec((blk, RH), lambda i: (i + P // blk, 0)),
            pl.BlockSpec((1, RH), lambda i: (0, 0)),
            pl.BlockSpec((RH, 1), lambda i: (0, 0)),
            pl.BlockSpec((1, 1), lambda i: (0, 0)),
        ],
        out_specs=pl.BlockSpec((blk, 1), lambda i: (i, 0)),
        out_shape=jax.ShapeDtypeStruct((P, 1), _f32),
    )(zg, zg, br1r, Wr2, br2r)


def kernel(x, edge_index, batch, drug_drug_batch,
           W1, b1, W2, b2, Wr1, br1, Wr2, br2):
    pad_e = N + jnp.arange(EP - E, dtype=jnp.int32) % (NP - N)
    src_f = jnp.concatenate([edge_index[0], pad_e])
    dst_f = jnp.concatenate([edge_index[1], pad_e])
    src = src_f.reshape(NW, ECW, CW)
    dst = dst_f.reshape(NW, ECW, CW)
    dsth = dst_f.reshape(NW * ECH, 128)
    bat = jnp.concatenate([batch, jnp.full((NP - N,), G, jnp.int32)])
    bat = bat.reshape(NW, NCH, 64)
    x_pad = jnp.pad(x, ((0, NP - N), (0, 0)))

    degf, cntf = _hist(dsth, bat)
    q1, dis = _tc1(x_pad, W1, degf.reshape(NC, NP, 1))
    acc1 = _prop(q1, src, dst)
    q2 = _tc2(acc1.reshape(NC, NP, D), q1, dis, b1.reshape(1, D), W2)
    acc2 = _prop(q2, src, dst)
    h = _tc3(acc2.reshape(NC, NP, D), q2, dis, b2.reshape(1, D))
    gs = _pool(h, bat)
    U, V = _tc4(gs.reshape(NC, GP, D), cntf.reshape(NC, GP, 1), Wr1)
    tab = jnp.concatenate([U, V], axis=0)
    pidx = jnp.concatenate([drug_drug_batch[0],
                            drug_drug_batch[1] + GP]).reshape(NW, 2, 128)
    zg = _pairs(tab, pidx)
    return _tc5(zg, br1.reshape(1, RH), Wr2, br2.reshape(1, 1))


# speedup vs baseline: 1.0229x; 1.0113x over previous
import jax
import jax.numpy as jnp
from jax import lax
from jax.experimental import pallas as pl
from jax.experimental.pallas import tpu as pltpu
from jax.experimental.pallas import tpu_sc as plsc

N = 10000
E = 320000
D = 128
G = 500
P = 4096
RH = 256

NC, NS = 2, 16
NW = NC * NS
NP = 10240
EP = 327680
GP = 512
ECH = 80
NCH = 5
RPT = NP // NS

_mesh = plsc.VectorSubcoreMesh(core_axis_name="c", subcore_axis_name="s")
_f32 = jnp.float32


def _zero_rows(ref, n16):
    z = jnp.zeros((16,), _f32)
    def row(i, _):
        for j in range(8):
            ref[i, pl.ds(j * 16, 16)] = z
        return 0
    lax.fori_loop(0, 16 * n16, row, 0)


def _fill_1d(ref, n, val):
    v = jnp.full((16,), val, _f32)
    def it(i, _):
        ref[pl.ds(i * 16, 16)] = v
        return 0
    lax.fori_loop(0, n // 16, it, 0)


def _hist_body(dst_hbm, bat_hbm, deg_out, cnt_out,
               didx_v, bidx_v, ones_v, zb_v, deg_sh, cnt_sh):
    cid = lax.axis_index("c")
    sid = lax.axis_index("s")
    wid = cid * NS + sid
    _fill_1d(ones_v, 128, 1.0)
    _fill_1d(zb_v, RPT, 0.0)
    pltpu.sync_copy(zb_v, deg_sh.at[pl.ds(sid * RPT, RPT)])
    pltpu.sync_copy(zb_v.at[pl.ds(0, GP // NS)],
                    cnt_sh.at[pl.ds(sid * (GP // NS), GP // NS)])
    plsc.subcore_barrier()
    pltpu.sync_copy(dst_hbm.at[pl.ds(wid * ECH, ECH)], didx_v)
    def echunk(c, _):
        pltpu.sync_copy(ones_v, deg_sh.at[didx_v.at[c]], add=True)
        return 0
    lax.fori_loop(0, ECH, echunk, 0)
    pltpu.sync_copy(bat_hbm.at[wid], bidx_v)
    def bchunk(c, _):
        pltpu.sync_copy(ones_v.at[pl.ds(0, 64)], cnt_sh.at[bidx_v.at[c]],
                        add=True)
        return 0
    lax.fori_loop(0, NCH, bchunk, 0)
    plsc.subcore_barrier()
    pltpu.sync_copy(deg_sh.at[pl.ds(sid * RPT, RPT)], zb_v)
    pltpu.sync_copy(zb_v, deg_out.at[pl.ds(cid * NP + sid * RPT, RPT)])
    pltpu.sync_copy(cnt_sh.at[pl.ds(sid * (GP // NS), GP // NS)],
                    ones_v.at[pl.ds(0, GP // NS)])
    pltpu.sync_copy(ones_v.at[pl.ds(0, GP // NS)],
                    cnt_out.at[pl.ds(cid * GP + sid * (GP // NS), GP // NS)])


_hist = pl.kernel(
    _hist_body,
    out_type=(jax.ShapeDtypeStruct((NC * NP,), _f32),
              jax.ShapeDtypeStruct((NC * GP,), _f32)),
    mesh=_mesh,
    scratch_types=[
        pltpu.VMEM((ECH, 128), jnp.int32),
        pltpu.VMEM((NCH, 64), jnp.int32),
        pltpu.VMEM((128,), _f32),
        pltpu.VMEM((RPT,), _f32),
        pltpu.VMEM_SHARED((NP,), _f32),
        pltpu.VMEM_SHARED((GP,), _f32),
    ],
)


CW = 64
NBUF = 4
NGIF = 2
ECW = EP // NW // CW
PH = 4
PCH = ECW // PH


def _prop_body(q_hbm, src_hbm, dst_hbm, out_hbm,
               sidx_v, didx_v, rows0, rows1, rows2, rows3, acc_sh,
               gs0, gs1, gs2, gs3, ss0, ss1, ss2, ss3):
    cid = lax.axis_index("c")
    sid = lax.axis_index("s")
    wid = cid * NS + sid
    rows = (rows0, rows1, rows2, rows3)
    gsem = (gs0, gs1, gs2, gs3)
    ssem = (ss0, ss1, ss2, ss3)
    _zero_rows(rows0, 1)
    def zrow(i, _):
        pltpu.sync_copy(rows0.at[pl.ds(0, 16)],
                        acc_sh.at[pl.ds(sid * RPT + i * 16, 16)])
        return 0
    lax.fori_loop(0, RPT // 16, zrow, 0)
    plsc.subcore_barrier()

    def wait_bytes(sem, buf):
        pltpu.make_async_copy(q_hbm.at[pl.ds(0, CW)], buf, sem).wait()

    def phase(p, _):
        pltpu.sync_copy(src_hbm.at[wid, pl.ds(p * PCH, PCH)], sidx_v)
        pltpu.sync_copy(dst_hbm.at[wid, pl.ds(p * PCH, PCH)], didx_v)
        for j in range(NGIF):
            pltpu.async_copy(q_hbm.at[sidx_v.at[j]], rows[j], gsem[j])
        def step(g, _):
            for b in range(NBUF):
                c = NBUF * g + b
                bj = (b + NGIF) % NBUF
                wait_bytes(gsem[b], rows[b])
                pltpu.async_copy(rows[b], acc_sh.at[didx_v.at[c]],
                                 ssem[b], add=True)
                @pl.when(c + NGIF < PCH)
                def _():
                    @pl.when(c >= NBUF - NGIF)
                    def _():
                        wait_bytes(ssem[bj], rows[bj])
                    pltpu.async_copy(q_hbm.at[sidx_v.at[c + NGIF]],
                                     rows[bj], gsem[bj])
            return 0
        lax.fori_loop(0, PCH // NBUF, step, 0)
        for b in range(NBUF):
            wait_bytes(ssem[b], rows[b])
        return 0
    lax.fori_loop(0, PH, phase, 0)
    plsc.subcore_barrier()
    def rb(i, _):
        pltpu.sync_copy(acc_sh.at[pl.ds(sid * RPT + i * CW, CW)], rows0)
        pltpu.sync_copy(rows0,
                        out_hbm.at[pl.ds(cid * NP + sid * RPT + i * CW, CW)])
        return 0
    lax.fori_loop(0, RPT // CW, rb, 0)


_prop = pl.kernel(
    _prop_body,
    out_type=jax.ShapeDtypeStruct((NC * NP, D), _f32),
    mesh=_mesh,
    scratch_types=[
        pltpu.VMEM((PCH, CW), jnp.int32),
        pltpu.VMEM((PCH, CW), jnp.int32),
        pltpu.VMEM((CW, D), _f32),
        pltpu.VMEM((CW, D), _f32),
        pltpu.VMEM((CW, D), _f32),
        pltpu.VMEM((CW, D), _f32),
        pltpu.VMEM_SHARED((NP, D), _f32),
        pltpu.SemaphoreType.DMA,
        pltpu.SemaphoreType.DMA,
        pltpu.SemaphoreType.DMA,
        pltpu.SemaphoreType.DMA,
        pltpu.SemaphoreType.DMA,
        pltpu.SemaphoreType.DMA,
        pltpu.SemaphoreType.DMA,
        pltpu.SemaphoreType.DMA,
    ],
)


def _pool_body(h_hbm, bat_hbm, out_hbm, bidx_v, rows_v, zr_v, g_sh):
    cid = lax.axis_index("c")
    sid = lax.axis_index("s")
    wid = cid * NS + sid
    _zero_rows(zr_v, 1)
    def zrow(i, _):
        pltpu.sync_copy(zr_v, g_sh.at[pl.ds(sid * (GP // NS) + i * 16, 16)])
        return 0
    lax.fori_loop(0, GP // NS // 16, zrow, 0)
    plsc.subcore_barrier()
    pltpu.sync_copy(bat_hbm.at[wid], bidx_v)
    def chunk(c, _):
        pltpu.sync_copy(h_hbm.at[pl.ds(wid * (NCH * 64) + c * 64, 64)], rows_v)
        pltpu.sync_copy(rows_v, g_sh.at[bidx_v.at[c]], add=True)
        return 0
    lax.fori_loop(0, NCH, chunk, 0)
    plsc.subcore_barrier()
    pltpu.sync_copy(g_sh.at[pl.ds(sid * (GP // NS), GP // NS)],
                    rows_v.at[pl.ds(0, GP // NS)])
    pltpu.sync_copy(rows_v.at[pl.ds(0, GP // NS)],
                    out_hbm.at[pl.ds(cid * GP + sid * (GP // NS), GP // NS)])


_pool = pl.kernel(
    _pool_body,
    out_type=jax.ShapeDtypeStruct((NC * GP, D), _f32),
    mesh=_mesh,
    scratch_types=[
        pltpu.VMEM((NCH, 64), jnp.int32),
        pltpu.VMEM((64, D), _f32),
        pltpu.VMEM((16, D), _f32),
        pltpu.VMEM_SHARED((GP, D), _f32),
    ],
)


def _pairs_body(tab_hbm, pidx_hbm, out_hbm, pidx_v, rows_v, sem):
    cid = lax.axis_index("c")
    sid = lax.axis_index("s")
    wid = cid * NS + sid
    pltpu.sync_copy(pidx_hbm.at[wid], pidx_v)
    def chunk(c, _):
        pltpu.async_copy(tab_hbm.at[pidx_v.at[c]], rows_v, sem).wait()
        pltpu.sync_copy(rows_v, out_hbm.at[pl.ds(wid * 256 + c * 128, 128)])
        return 0
    lax.fori_loop(0, 2, chunk, 0)


_pairs = pl.kernel(
    _pairs_body,
    out_type=jax.ShapeDtypeStruct((2 * P, RH), _f32),
    mesh=_mesh,
    scratch_types=[
        pltpu.VMEM((2, 128), jnp.int32),
        pltpu.VMEM((128, RH), _f32),
        pltpu.SemaphoreType.DMA,
    ],
)


_RB = 1024


def _tc1_body(x_ref, w_ref, deg_ref, q_ref, dis_ref):
    deg = deg_ref[0] + deg_ref[1] + 1.0
    dis = lax.rsqrt(deg)
    h = jnp.dot(x_ref[...], w_ref[...], preferred_element_type=_f32)
    q_ref[...] = h * dis
    dis_ref[...] = dis


def _tc1(x_pad, W1, deg3):
    return pl.pallas_call(
        _tc1_body,
        grid=(NP // _RB,),
        in_specs=[
            pl.BlockSpec((_RB, D), lambda i: (i, 0)),
            pl.BlockSpec((D, D), lambda i: (0, 0)),
            pl.BlockSpec((NC, _RB, 1), lambda i: (0, i, 0)),
        ],
        out_specs=[
            pl.BlockSpec((_RB, D), lambda i: (i, 0)),
            pl.BlockSpec((_RB, 1), lambda i: (i, 0)),
        ],
        out_shape=[jax.ShapeDtypeStruct((NP, D), _f32),
                   jax.ShapeDtypeStruct((NP, 1), _f32)],
    )(x_pad, W1, deg3)


def _tc2_body(acc_ref, q_ref, dis_ref, b_ref, w_ref, oq_ref):
    s = acc_ref[0] + acc_ref[1] + q_ref[...]
    o1 = jnp.maximum(s * dis_ref[...] + b_ref[...], 0.0)
    oq_ref[...] = jnp.dot(o1, w_ref[...],
                          preferred_element_type=_f32) * dis_ref[...]


def _tc2(acc3, q1, dis, b1r, W2):
    return pl.pallas_call(
        _tc2_body,
        grid=(NP // _RB,),
        in_specs=[
            pl.BlockSpec((NC, _RB, D), lambda i: (0, i, 0)),
            pl.BlockSpec((_RB, D), lambda i: (i, 0)),
            pl.BlockSpec((_RB, 1), lambda i: (i, 0)),
            pl.BlockSpec((1, D), lambda i: (0, 0)),
            pl.BlockSpec((D, D), lambda i: (0, 0)),
        ],
        out_specs=pl.BlockSpec((_RB, D), lambda i: (i, 0)),
        out_shape=jax.ShapeDtypeStruct((NP, D), _f32),
    )(acc3, q1, dis, b1r, W2)


def _tc3_body(acc_ref, q_ref, dis_ref, b_ref, h_ref):
    s = acc_ref[0] + acc_ref[1] + q_ref[...]
    h_ref[...] = jnp.maximum(s * dis_ref[...] + b_ref[...], 0.0)


def _tc3(acc3, q2, dis, b2r):
    return pl.pallas_call(
        _tc3_body,
        grid=(NP // _RB,),
        in_specs=[
            pl.BlockSpec((NC, _RB, D), lambda i: (0, i, 0)),
            pl.BlockSpec((_RB, D), lambda i: (i, 0)),
            pl.BlockSpec((_RB, 1), lambda i: (i, 0)),
            pl.BlockSpec((1, D), lambda i: (0, 0)),
        ],
        out_specs=pl.BlockSpec((_RB, D), lambda i: (i, 0)),
        out_shape=jax.ShapeDtypeStruct((NP, D), _f32),
    )(acc3, q2, dis, b2r)


def _tc4_body(gs_ref, cnt_ref, w_ref, u_ref, v_ref):
    cnt = cnt_ref[0] + cnt_ref[1]
    ge = (gs_ref[0] + gs_ref[1]) / jnp.maximum(cnt, 1.0)
    u_ref[...] = jnp.dot(ge, w_ref[0:D, :], preferred_element_type=_f32)
    v_ref[...] = jnp.dot(ge, w_ref[D:2 * D, :], preferred_element_type=_f32)


def _tc4(gs3, cnt3, Wr1):
    return pl.pallas_call(
        _tc4_body,
        out_shape=[jax.ShapeDtypeStruct((GP, RH), _f32),
                   jax.ShapeDtypeStruct((GP, RH), _f32)],
    )(gs3, cnt3, Wr1)


def _tc5_body(zf_ref, zt_ref, b1_ref, w_ref, b2_ref, o_ref):
    o = jnp.maximum(zf_ref[...] + zt_ref[...] + b1_ref[...], 0.0)
    o_ref[...] = jnp.dot(o, w_ref[...], preferred_element_type=_f32) + b2_ref[...]


def _tc5(zg, br1r, Wr2, br2r):
    blk = 512
    return pl.pallas_call(
        _tc5_body,
        grid=(P // blk,),
        in_specs=[
            pl.BlockSpec((blk, RH), lambda i: (i, 0)),
            pl.BlockSpec((blk, RH), lambda i: (i + P // blk, 0)),
            pl.BlockSpec((1, RH), lambda i: (0, 0)),
            pl.BlockSpec((RH, 1), lambda i: (0, 0)),
            pl.BlockSpec((1, 1), lambda i: (0, 0)),
        ],
        out_specs=pl.BlockSpec((blk, 1), lambda i: (i, 0)),
        out_shape=jax.ShapeDtypeStruct((P, 1), _f32),
    )(zg, zg, br1r, Wr2, br2r)


def kernel(x, edge_index, batch, drug_drug_batch,
           W1, b1, W2, b2, Wr1, br1, Wr2, br2):
    pad_e = N + jnp.arange(EP - E, dtype=jnp.int32) % (NP - N)
    src_f = jnp.concatenate([edge_index[0], pad_e])
    dst_f = jnp.concatenate([edge_index[1], pad_e])
    src = src_f.reshape(NW, ECW, CW)
    dst = dst_f.reshape(NW, ECW, CW)
    dsth = dst_f.reshape(NW * ECH, 128)
    bat = jnp.concatenate([batch, jnp.full((NP - N,), G, jnp.int32)])
    bat = bat.reshape(NW, NCH, 64)
    x_pad = jnp.pad(x, ((0, NP - N), (0, 0)))

    degf, cntf = _hist(dsth, bat)
    q1, dis = _tc1(x_pad, W1, degf.reshape(NC, NP, 1))
    acc1 = _prop(q1, src, dst)
    q2 = _tc2(acc1.reshape(NC, NP, D), q1, dis, b1.reshape(1, D), W2)
    acc2 = _prop(q2, src, dst)
    h = _tc3(acc2.reshape(NC, NP, D), q2, dis, b2.reshape(1, D))
    gs = _pool(h, bat)
    U, V = _tc4(gs.reshape(NC, GP, D), cntf.reshape(NC, GP, 1), Wr1)
    tab = jnp.concatenate([U, V], axis=0)
    pidx = jnp.concatenate([drug_drug_batch[0],
                            drug_drug_batch[1] + GP]).reshape(NW, 2, 128)
    zg = _pairs(tab, pidx)
    return _tc5(zg, br1.reshape(1, RH), Wr2, br2.reshape(1, 1))
